# baseline (speedup 1.0000x reference)
import functools

import jax
import jax.numpy as jnp
from jax import lax
from jax.experimental import pallas as pl
from jax.experimental.pallas import tpu as pltpu
from jax.experimental.pallas import tpu_sc as plsc

E = 8
TN = 384
NBLK = 4096 // TN + E
RCH = 256
NC, NS, NW = 2, 16, 32


def _router_kernel(x_ref, rw_ref, pos_ref, wtsx_ref, bexp_ref, nact_ref,
                   aux_ref):
    n = x_ref.shape[0]
    xf = x_ref[...]
    logits = lax.dot_general(xf, rw_ref[...], (((1,), (1,)), ((), ())))
    m = jnp.max(logits, -1, keepdims=True)
    p = jnp.exp(logits - m)
    probs = p / jnp.sum(p, -1, keepdims=True)
    iota = lax.broadcasted_iota(jnp.int32, probs.shape, 1)
    m1 = jnp.max(probs, -1, keepdims=True)
    i1 = jnp.min(jnp.where(probs == m1, iota, E), -1, keepdims=True)
    mask1 = iota == i1
    pm = jnp.where(mask1, -jnp.inf, probs)
    m2 = jnp.max(pm, -1, keepdims=True)
    i2 = jnp.min(jnp.where(pm == m2, iota, E), -1, keepdims=True)
    mask2 = iota == i2
    denom = m1 + m2 + 1e-9
    wtsx_ref[0] = jnp.broadcast_to(m1 / denom, (n, 16))
    wtsx_ref[1] = jnp.broadcast_to(m2 / denom, (n, 16))

    oh1 = mask1.astype(jnp.float32)
    oh = oh1 + mask2.astype(jnp.float32)
    c = jnp.sum(oh, axis=0, keepdims=True)
    nb = jnp.floor((c + (TN - 1)) * (1.0 / TN))
    er = lax.broadcasted_iota(jnp.int32, (E, E), 0)
    ec = lax.broadcasted_iota(jnp.int32, (E, E), 1)
    t_excl = (er < ec).astype(jnp.float32)
    off_row = lax.dot_general(nb, t_excl,
                              (((1,), (0,)), ((), ()))) * TN

    ones_col = jnp.ones((n, 1), jnp.float32)
    c_col = lax.dot_general(oh, ones_col, (((0,), (0,)), ((), ())))
    nb_col = jnp.floor((c_col + (TN - 1)) * (1.0 / TN))
    t_incl = (ec <= er).astype(jnp.float32)
    po_col = lax.dot_general(t_incl, nb_col, (((1,), (0,)), ((), ())))
    giota = lax.broadcasted_iota(jnp.int32, (E, NBLK), 1)
    bexp_i = jnp.sum((giota >= po_col.astype(jnp.int32)).astype(jnp.int32),
                     axis=0, keepdims=True)
    bexp_ref[...] = jnp.minimum(bexp_i, E - 1)
    nact_ref[...] = jnp.reshape(jnp.max(po_col), (1, 1)).astype(jnp.int32)

    rr = lax.broadcasted_iota(jnp.int32, (RCH, RCH), 0)
    cc = lax.broadcasted_iota(jnp.int32, (RCH, RCH), 1)
    t_strict = (cc < rr).astype(jnp.float32)
    run = jnp.zeros((1, E), jnp.float32)
    for i in range(n // RCH):
        sl = slice(i * RCH, (i + 1) * RCH)
        oh_c = oh[sl]
        cnt = run + lax.dot_general(t_strict, oh_c,
                                    (((1,), (0,)), ((), ())))
        run = run + jnp.sum(oh_c, axis=0, keepdims=True)
        dest = off_row + cnt
        p0 = jnp.sum(jnp.where(mask1[sl], dest, 0.0), -1, keepdims=True)
        p1 = jnp.sum(jnp.where(mask2[sl], dest, 0.0), -1, keepdims=True)
        pos_ref[sl, :] = jnp.concatenate([p0, p1], 1).astype(jnp.int32)

    importance = jnp.sum(probs, axis=0) / n
    load = jnp.sum(oh1, axis=0) / n
    aux_ref[...] = jnp.reshape(jnp.sum(importance * load) * E, (1, 1))


KS = 1


def _expert_kernel(bexp_ref, nact_ref, xs_ref, w1_ref, b1_ref,
                   w2_ref, b2_ref, ys_ref):
    g = pl.program_id(0)
    k = pl.program_id(1)
    active = g < nact_ref[0, 0]

    @pl.when(active)
    def _():
        xb = xs_ref[...]
        w1 = w1_ref[0].astype(jnp.bfloat16)
        h = lax.dot_general(xb, w1, (((1,), (1,)), ((), ())),
                            preferred_element_type=jnp.float32) + b1_ref[0]
        h = h * jax.nn.sigmoid(h)
        w2 = w2_ref[0].astype(jnp.bfloat16)
        part = lax.dot_general(h.astype(jnp.bfloat16), w2,
                               (((1,), (1,)), ((), ())),
                               preferred_element_type=jnp.float32)

        @pl.when(k == 0)
        def _init():
            ys_ref[...] = part + b2_ref[0]

        @pl.when(k > 0)
        def _acc():
            ys_ref[...] += part


def _make_scatter(n, d, nrows, tpw):
    mesh = plsc.VectorSubcoreMesh(core_axis_name="c", subcore_axis_name="s")

    @functools.partial(
        pl.kernel, mesh=mesh,
        out_type=jax.ShapeDtypeStruct((nrows, d), jnp.float32),
        scratch_types=[pltpu.VMEM((tpw, d), jnp.float32),
                       pltpu.VMEM((tpw,), jnp.int32),
                       pltpu.VMEM((tpw,), jnp.int32),
                       pltpu.SemaphoreType.DMA],
    )
    def scatter_x(x_hbm, pos_hbm, xs_hbm, xv, i0, i1, sem):
        wid = lax.axis_index("s") * NC + lax.axis_index("c")
        base = wid * tpw
        pltpu.sync_copy(x_hbm.at[pl.ds(base, tpw)], xv)
        pltpu.sync_copy(pos_hbm.at[0, pl.ds(base, tpw)], i0)
        pltpu.sync_copy(pos_hbm.at[1, pl.ds(base, tpw)], i1)
        cp0 = pltpu.make_async_copy(xv, xs_hbm.at[i0], sem)
        cp1 = pltpu.make_async_copy(xv, xs_hbm.at[i1], sem)
        cp0.start()
        cp1.start()
        cp0.wait()
        cp1.wait()

    return scatter_x


def _make_combine(n, d, nrows, tpw):
    mesh = plsc.VectorSubcoreMesh(core_axis_name="c", subcore_axis_name="s")

    @functools.partial(
        pl.kernel, mesh=mesh,
        out_type=jax.ShapeDtypeStruct((n, d), jnp.float32),
        scratch_types=[pltpu.VMEM((tpw, d), jnp.float32),
                       pltpu.VMEM((tpw, d), jnp.float32),
                       pltpu.VMEM((tpw,), jnp.int32),
                       pltpu.VMEM((tpw,), jnp.int32),
                       pltpu.VMEM((tpw, 16), jnp.float32),
                       pltpu.VMEM((tpw, 16), jnp.float32),
                       pltpu.SemaphoreType.DMA],
    )
    def combine(ys_hbm, pos_hbm, wts_hbm, y_hbm, ra, rb, i0, i1, w0, w1, sem):
        wid = lax.axis_index("s") * NC + lax.axis_index("c")
        base = wid * tpw
        pltpu.sync_copy(pos_hbm.at[0, pl.ds(base, tpw)], i0)
        pltpu.sync_copy(pos_hbm.at[1, pl.ds(base, tpw)], i1)
        pltpu.sync_copy(wts_hbm.at[0, pl.ds(base, tpw)], w0)
        pltpu.sync_copy(wts_hbm.at[1, pl.ds(base, tpw)], w1)
        cp0 = pltpu.make_async_copy(ys_hbm.at[i0], ra, sem)
        cp1 = pltpu.make_async_copy(ys_hbm.at[i1], rb, sem)
        cp0.start()
        cp1.start()
        cp0.wait()
        cp1.wait()

        def tbody(t, _):
            s0 = w0[t, :]
            s1 = w1[t, :]
            for ci in range(d // 16):
                sl = pl.ds(ci * 16, 16)
                ra[t, sl] = s0 * ra[t, sl] + s1 * rb[t, sl]
            return 0

        lax.fori_loop(0, tpw, tbody, 0)
        pltpu.sync_copy(ra, y_hbm.at[pl.ds(base, tpw)])

    return combine


def kernel(x, router_W, fc1_W, fc1_b, fc2_W, fc2_b):
    B, L, D = x.shape
    N = B * L
    H = fc1_W.shape[1]
    NROWS = NBLK * TN
    TPW = N // NW
    x_flat = x.reshape(N, D)

    pos, wtsx, bexp, nact, aux = pl.pallas_call(
        _router_kernel,
        out_shape=[
            jax.ShapeDtypeStruct((N, 2), jnp.int32),
            jax.ShapeDtypeStruct((2, N, 16), jnp.float32),
            jax.ShapeDtypeStruct((1, NBLK), jnp.int32),
            jax.ShapeDtypeStruct((1, 1), jnp.int32),
            jax.ShapeDtypeStruct((1, 1), jnp.float32),
        ],
    )(x_flat, router_W)

    pos_t = pos.T

    xs = _make_scatter(N, D, NROWS, TPW)(x_flat, pos_t)

    grid_spec = pltpu.PrefetchScalarGridSpec(
        num_scalar_prefetch=2,
        grid=(NBLK, KS),
        in_specs=[
            pl.BlockSpec((TN, D),
                         lambda g, k, be, na: (jnp.minimum(g, na[0, 0] - 1),
                                               0)),
            pl.BlockSpec((1, H // KS, D),
                         lambda g, k, be, na: (be[0, g], k, 0)),
            pl.BlockSpec((1, 1, H // KS),
                         lambda g, k, be, na: (be[0, g], 0, k)),
            pl.BlockSpec((1, D, H // KS),
                         lambda g, k, be, na: (be[0, g], 0, k)),
            pl.BlockSpec((1, 1, D), lambda g, k, be, na: (be[0, g], 0, 0)),
        ],
        out_specs=pl.BlockSpec(
            (TN, D),
            lambda g, k, be, na: (jnp.minimum(g, na[0, 0] - 1), 0)),
    )
    ys = pl.pallas_call(
        _expert_kernel,
        grid_spec=grid_spec,
        out_shape=jax.ShapeDtypeStruct((NROWS, D), jnp.float32),
    )(bexp, nact, xs, fc1_W, fc1_b.reshape(E, 1, H),
      fc2_W, fc2_b.reshape(E, 1, D))

    y = _make_combine(N, D, NROWS, TPW)(ys, pos_t, wtsx)
    return y.reshape(B, L, D), aux[0, 0]


# speedup vs baseline: 1.0752x; 1.0752x over previous
import functools

import jax
import jax.numpy as jnp
from jax import lax
from jax.experimental import pallas as pl
from jax.experimental.pallas import tpu as pltpu
from jax.experimental.pallas import tpu_sc as plsc

E = 8
TN = 512
NBLK = 4096 // TN + E
RCH = 256
NC, NS, NW = 2, 16, 32


def _router_kernel(x_ref, rw_ref, pos_ref, wtsx_ref, bexp_ref, nact_ref,
                   aux_ref):
    n = x_ref.shape[0]
    xf = x_ref[...]
    logits = lax.dot_general(xf, rw_ref[...], (((1,), (1,)), ((), ())))
    m = jnp.max(logits, -1, keepdims=True)
    p = jnp.exp(logits - m)
    probs = p / jnp.sum(p, -1, keepdims=True)
    iota = lax.broadcasted_iota(jnp.int32, probs.shape, 1)
    m1 = jnp.max(probs, -1, keepdims=True)
    i1 = jnp.min(jnp.where(probs == m1, iota, E), -1, keepdims=True)
    mask1 = iota == i1
    pm = jnp.where(mask1, -jnp.inf, probs)
    m2 = jnp.max(pm, -1, keepdims=True)
    i2 = jnp.min(jnp.where(pm == m2, iota, E), -1, keepdims=True)
    mask2 = iota == i2
    denom = m1 + m2 + 1e-9
    wtsx_ref[0] = jnp.broadcast_to(m1 / denom, (n, 16))
    wtsx_ref[1] = jnp.broadcast_to(m2 / denom, (n, 16))

    oh1 = mask1.astype(jnp.float32)
    oh = oh1 + mask2.astype(jnp.float32)
    c = jnp.sum(oh, axis=0, keepdims=True)
    nb = jnp.floor((c + (TN - 1)) * (1.0 / TN))
    er = lax.broadcasted_iota(jnp.int32, (E, E), 0)
    ec = lax.broadcasted_iota(jnp.int32, (E, E), 1)
    t_excl = (er < ec).astype(jnp.float32)
    off_row = lax.dot_general(nb, t_excl,
                              (((1,), (0,)), ((), ()))) * TN

    ones_col = jnp.ones((n, 1), jnp.float32)
    c_col = lax.dot_general(oh, ones_col, (((0,), (0,)), ((), ())))
    nb_col = jnp.floor((c_col + (TN - 1)) * (1.0 / TN))
    t_incl = (ec <= er).astype(jnp.float32)
    po_col = lax.dot_general(t_incl, nb_col, (((1,), (0,)), ((), ())))
    giota = lax.broadcasted_iota(jnp.int32, (E, NBLK), 1)
    bexp_i = jnp.sum((giota >= po_col.astype(jnp.int32)).astype(jnp.int32),
                     axis=0, keepdims=True)
    bexp_ref[...] = jnp.minimum(bexp_i, E - 1)
    nact_ref[...] = jnp.reshape(jnp.max(po_col), (1, 1)).astype(jnp.int32)

    rr = lax.broadcasted_iota(jnp.int32, (RCH, RCH), 0)
    cc = lax.broadcasted_iota(jnp.int32, (RCH, RCH), 1)
    t_strict = (cc < rr).astype(jnp.float32)
    run = jnp.zeros((1, E), jnp.float32)
    for i in range(n // RCH):
        sl = slice(i * RCH, (i + 1) * RCH)
        oh_c = oh[sl]
        cnt = run + lax.dot_general(t_strict, oh_c,
                                    (((1,), (0,)), ((), ())))
        run = run + jnp.sum(oh_c, axis=0, keepdims=True)
        dest = off_row + cnt
        p0 = jnp.sum(jnp.where(mask1[sl], dest, 0.0), -1, keepdims=True)
        p1 = jnp.sum(jnp.where(mask2[sl], dest, 0.0), -1, keepdims=True)
        pos_ref[sl, :] = jnp.concatenate([p0, p1], 1).astype(jnp.int32)

    importance = jnp.sum(probs, axis=0) / n
    load = jnp.sum(oh1, axis=0) / n
    aux_ref[...] = jnp.reshape(jnp.sum(importance * load) * E, (1, 1))


KS = 1


def _expert_kernel(bexp_ref, nact_ref, xs_ref, w1_ref, b1_ref,
                   w2_ref, b2_ref, ys_ref):
    g = pl.program_id(0)
    k = pl.program_id(1)
    active = g < nact_ref[0, 0]

    @pl.when(active)
    def _():
        xb = xs_ref[...]
        w1 = w1_ref[0].astype(jnp.bfloat16)
        h = lax.dot_general(xb, w1, (((1,), (1,)), ((), ())),
                            preferred_element_type=jnp.float32) + b1_ref[0]
        h = h * jax.nn.sigmoid(h)
        w2 = w2_ref[0].astype(jnp.bfloat16)
        part = lax.dot_general(h.astype(jnp.bfloat16), w2,
                               (((1,), (1,)), ((), ())),
                               preferred_element_type=jnp.float32)

        @pl.when(k == 0)
        def _init():
            ys_ref[...] = part + b2_ref[0]

        @pl.when(k > 0)
        def _acc():
            ys_ref[...] += part


def _make_scatter(n, d, nrows, tpw):
    mesh = plsc.VectorSubcoreMesh(core_axis_name="c", subcore_axis_name="s")

    @functools.partial(
        pl.kernel, mesh=mesh,
        out_type=jax.ShapeDtypeStruct((nrows, d), jnp.float32),
        scratch_types=[pltpu.VMEM((tpw, d), jnp.float32),
                       pltpu.VMEM((tpw,), jnp.int32),
                       pltpu.VMEM((tpw,), jnp.int32),
                       pltpu.SemaphoreType.DMA],
    )
    def scatter_x(x_hbm, pos_hbm, xs_hbm, xv, i0, i1, sem):
        wid = lax.axis_index("s") * NC + lax.axis_index("c")
        base = wid * tpw
        pltpu.sync_copy(x_hbm.at[pl.ds(base, tpw)], xv)
        pltpu.sync_copy(pos_hbm.at[0, pl.ds(base, tpw)], i0)
        pltpu.sync_copy(pos_hbm.at[1, pl.ds(base, tpw)], i1)
        cp0 = pltpu.make_async_copy(xv, xs_hbm.at[i0], sem)
        cp1 = pltpu.make_async_copy(xv, xs_hbm.at[i1], sem)
        cp0.start()
        cp1.start()
        cp0.wait()
        cp1.wait()

    return scatter_x


def _make_combine(n, d, nrows, tpw):
    mesh = plsc.VectorSubcoreMesh(core_axis_name="c", subcore_axis_name="s")

    @functools.partial(
        pl.kernel, mesh=mesh,
        out_type=jax.ShapeDtypeStruct((n, d), jnp.float32),
        scratch_types=[pltpu.VMEM((tpw, d), jnp.float32),
                       pltpu.VMEM((tpw, d), jnp.float32),
                       pltpu.VMEM((tpw,), jnp.int32),
                       pltpu.VMEM((tpw,), jnp.int32),
                       pltpu.VMEM((tpw, 16), jnp.float32),
                       pltpu.VMEM((tpw, 16), jnp.float32),
                       pltpu.SemaphoreType.DMA],
    )
    def combine(ys_hbm, pos_hbm, wts_hbm, y_hbm, ra, rb, i0, i1, w0, w1, sem):
        wid = lax.axis_index("s") * NC + lax.axis_index("c")
        base = wid * tpw
        pltpu.sync_copy(pos_hbm.at[0, pl.ds(base, tpw)], i0)
        pltpu.sync_copy(pos_hbm.at[1, pl.ds(base, tpw)], i1)
        pltpu.sync_copy(wts_hbm.at[0, pl.ds(base, tpw)], w0)
        pltpu.sync_copy(wts_hbm.at[1, pl.ds(base, tpw)], w1)
        cp0 = pltpu.make_async_copy(ys_hbm.at[i0], ra, sem)
        cp1 = pltpu.make_async_copy(ys_hbm.at[i1], rb, sem)
        cp0.start()
        cp1.start()
        cp0.wait()
        cp1.wait()

        def tbody(t, _):
            s0 = w0[t, :]
            s1 = w1[t, :]
            for ci in range(d // 16):
                sl = pl.ds(ci * 16, 16)
                ra[t, sl] = s0 * ra[t, sl] + s1 * rb[t, sl]
            return 0

        lax.fori_loop(0, tpw, tbody, 0)
        pltpu.sync_copy(ra, y_hbm.at[pl.ds(base, tpw)])

    return combine


def kernel(x, router_W, fc1_W, fc1_b, fc2_W, fc2_b):
    B, L, D = x.shape
    N = B * L
    H = fc1_W.shape[1]
    NROWS = NBLK * TN
    TPW = N // NW
    x_flat = x.reshape(N, D)

    pos, wtsx, bexp, nact, aux = pl.pallas_call(
        _router_kernel,
        out_shape=[
            jax.ShapeDtypeStruct((N, 2), jnp.int32),
            jax.ShapeDtypeStruct((2, N, 16), jnp.float32),
            jax.ShapeDtypeStruct((1, NBLK), jnp.int32),
            jax.ShapeDtypeStruct((1, 1), jnp.int32),
            jax.ShapeDtypeStruct((1, 1), jnp.float32),
        ],
    )(x_flat, router_W)

    pos_t = pos.T

    xs = _make_scatter(N, D, NROWS, TPW)(x_flat, pos_t)

    grid_spec = pltpu.PrefetchScalarGridSpec(
        num_scalar_prefetch=2,
        grid=(NBLK, KS),
        in_specs=[
            pl.BlockSpec((TN, D),
                         lambda g, k, be, na: (jnp.minimum(g, na[0, 0] - 1),
                                               0)),
            pl.BlockSpec((1, H // KS, D),
                         lambda g, k, be, na: (be[0, g], k, 0)),
            pl.BlockSpec((1, 1, H // KS),
                         lambda g, k, be, na: (be[0, g], 0, k)),
            pl.BlockSpec((1, D, H // KS),
                         lambda g, k, be, na: (be[0, g], 0, k)),
            pl.BlockSpec((1, 1, D), lambda g, k, be, na: (be[0, g], 0, 0)),
        ],
        out_specs=pl.BlockSpec(
            (TN, D),
            lambda g, k, be, na: (jnp.minimum(g, na[0, 0] - 1), 0)),
    )
    ys = pl.pallas_call(
        _expert_kernel,
        grid_spec=grid_spec,
        out_shape=jax.ShapeDtypeStruct((NROWS, D), jnp.float32),
    )(bexp, nact, xs, fc1_W, fc1_b.reshape(E, 1, H),
      fc2_W, fc2_b.reshape(E, 1, D))

    y = _make_combine(N, D, NROWS, TPW)(ys, pos_t, wtsx)
    return y.reshape(B, L, D), aux[0, 0]
